# Initial kernel scaffold; baseline (speedup 1.0000x reference)
import jax
import jax.numpy as jnp
from jax.experimental import pallas as pl


def kernel(Path, node2vec):
    raise NotImplementedError("write your pallas kernel here")


# speedup vs baseline: 4.0579x; 4.0579x over previous
import functools

import jax
import jax.numpy as jnp
from jax import lax
from jax.experimental import pallas as pl
from jax.experimental.pallas import tpu as pltpu
from jax.experimental.pallas import tpu_sc as plsc

EMBED_DIM = 64
NUM_WORKERS = 32
CHUNK = 128
N_CHUNKS = 50
B_PER_W = CHUNK * N_CHUNKS
TOTAL = B_PER_W * NUM_WORKERS


@functools.partial(
    pl.kernel,
    mesh=plsc.VectorSubcoreMesh(core_axis_name="c", subcore_axis_name="s"),
    out_type=jax.ShapeDtypeStruct((TOTAL, EMBED_DIM), jnp.float32),
    scratch_types=[
        pltpu.VMEM((N_CHUNKS, CHUNK), jnp.int32),
        pltpu.VMEM((CHUNK, EMBED_DIM), jnp.float32),
        pltpu.SemaphoreType.DMA,
    ],
    compiler_params=pltpu.CompilerParams(use_tc_tiling_on_sc=False),
)
def _gather(idx_hbm, table_hbm, out_hbm, idx_v, rows_v, sem):
    wid = lax.axis_index("s") * 2 + lax.axis_index("c")
    base = wid * B_PER_W
    pltpu.sync_copy(idx_hbm.at[wid], idx_v)
    for c in range(N_CHUNKS):
        pltpu.async_copy(table_hbm.at[idx_v.at[c]], rows_v, sem).wait()
        pltpu.sync_copy(rows_v, out_hbm.at[pl.ds(base + c * CHUNK, CHUNK)])


def kernel(Path, node2vec):
    B, H = Path.shape
    idx = Path.reshape(NUM_WORKERS, N_CHUNKS, CHUNK).astype(jnp.int32)
    out = _gather(idx, node2vec)
    return out.reshape(B, H, EMBED_DIM)


# speedup vs baseline: 4.6283x; 1.1406x over previous
import functools

import jax
import jax.numpy as jnp
from jax import lax
from jax.experimental import pallas as pl
from jax.experimental.pallas import tpu as pltpu
from jax.experimental.pallas import tpu_sc as plsc

EMBED_DIM = 64
NUM_WORKERS = 32
CHUNK = 128
N_CHUNKS = 50
B_PER_W = CHUNK * N_CHUNKS
TOTAL = B_PER_W * NUM_WORKERS
NBUF = 6
LAG = 3


@functools.partial(
    pl.kernel,
    mesh=plsc.VectorSubcoreMesh(core_axis_name="c", subcore_axis_name="s"),
    out_type=jax.ShapeDtypeStruct((TOTAL, EMBED_DIM), jnp.float32),
    scratch_types=[
        pltpu.VMEM((N_CHUNKS, CHUNK), jnp.int32),
        pltpu.VMEM((NBUF, CHUNK, EMBED_DIM), jnp.float32),
        pltpu.SemaphoreType.DMA((NBUF,)),
        pltpu.SemaphoreType.DMA((NBUF,)),
    ],
    compiler_params=pltpu.CompilerParams(use_tc_tiling_on_sc=False),
)
def _gather(idx_hbm, table_hbm, out_hbm, idx_v, rows_v, gsem, wsem):
    wid = lax.axis_index("s") * 2 + lax.axis_index("c")
    base = wid * B_PER_W
    pltpu.sync_copy(idx_hbm.at[wid], idx_v)

    gd = [None] * N_CHUNKS
    wd = [None] * N_CHUNKS

    def start_write(j):
        gd[j].wait()
        b = j % NBUF
        wd[j] = pltpu.async_copy(
            rows_v.at[b], out_hbm.at[pl.ds(base + j * CHUNK, CHUNK)], wsem.at[b]
        )

    for c in range(N_CHUNKS):
        b = c % NBUF
        if c >= NBUF:
            wd[c - NBUF].wait()
        gd[c] = pltpu.async_copy(table_hbm.at[idx_v.at[c]], rows_v.at[b], gsem.at[b])
        if c >= LAG:
            start_write(c - LAG)
    for j in range(N_CHUNKS - LAG, N_CHUNKS):
        start_write(j)
    for j in range(N_CHUNKS - NBUF, N_CHUNKS):
        wd[j].wait()


def kernel(Path, node2vec):
    B, H = Path.shape
    idx = Path.reshape(NUM_WORKERS, N_CHUNKS, CHUNK).astype(jnp.int32)
    out = _gather(idx, node2vec)
    return out.reshape(B, H, EMBED_DIM)


# speedup vs baseline: 4.6688x; 1.0087x over previous
import functools

import jax
import jax.numpy as jnp
from jax import lax
from jax.experimental import pallas as pl
from jax.experimental.pallas import tpu as pltpu
from jax.experimental.pallas import tpu_sc as plsc

EMBED_DIM = 64
NUM_WORKERS = 32
CHUNK = 800
N_CHUNKS = 8
B_PER_W = CHUNK * N_CHUNKS
TOTAL = B_PER_W * NUM_WORKERS
NBUF = 2
LAG = 1


@functools.partial(
    pl.kernel,
    mesh=plsc.VectorSubcoreMesh(core_axis_name="c", subcore_axis_name="s"),
    out_type=jax.ShapeDtypeStruct((TOTAL, EMBED_DIM), jnp.float32),
    scratch_types=[
        pltpu.VMEM((N_CHUNKS, CHUNK), jnp.int32),
        pltpu.VMEM((NBUF, CHUNK, EMBED_DIM), jnp.float32),
        pltpu.SemaphoreType.DMA((NBUF,)),
        pltpu.SemaphoreType.DMA((NBUF,)),
    ],
    compiler_params=pltpu.CompilerParams(use_tc_tiling_on_sc=False),
)
def _gather(idx_hbm, table_hbm, out_hbm, idx_v, rows_v, gsem, wsem):
    wid = lax.axis_index("s") * 2 + lax.axis_index("c")
    base = wid * B_PER_W
    pltpu.sync_copy(idx_hbm.at[wid], idx_v)

    gd = [None] * N_CHUNKS
    wd = [None] * N_CHUNKS

    def start_write(j):
        gd[j].wait()
        b = j % NBUF
        wd[j] = pltpu.async_copy(
            rows_v.at[b], out_hbm.at[pl.ds(base + j * CHUNK, CHUNK)], wsem.at[b]
        )

    for c in range(N_CHUNKS):
        b = c % NBUF
        if c >= NBUF:
            wd[c - NBUF].wait()
        gd[c] = pltpu.async_copy(table_hbm.at[idx_v.at[c]], rows_v.at[b], gsem.at[b])
        if c >= LAG:
            start_write(c - LAG)
    for j in range(N_CHUNKS - LAG, N_CHUNKS):
        start_write(j)
    for j in range(N_CHUNKS - NBUF, N_CHUNKS):
        wd[j].wait()


def kernel(Path, node2vec):
    B, H = Path.shape
    idx = Path.reshape(NUM_WORKERS, N_CHUNKS, CHUNK).astype(jnp.int32)
    out = _gather(idx, node2vec)
    return out.reshape(B, H, EMBED_DIM)
